# baseline (speedup 1.0000x reference)
import functools

import jax
import jax.numpy as jnp
import numpy as np
from jax import lax
from jax.experimental import pallas as pl
from jax.experimental.pallas import tpu as pltpu
from jax.experimental.pallas import tpu_sc as plsc

_DIMS = np.array(
    [6, 7, 8, 9, 10, 11, 12, 13, 14, 15, 16, 17, 21, 22, 23, 24, 25, 26,
     27, 28, 29, 30, 31, 32, 36, 37, 38, 39, 40, 41, 42, 43, 44, 45, 46,
     47, 51, 52, 53, 54, 55, 56, 57, 58, 59, 63, 64, 65, 66, 67, 68, 75,
     76, 77, 78, 79, 80, 81, 82, 83, 87, 88, 89, 90, 91, 92],
    dtype=np.int64,
)

_IN_W = 96
_OUT_W = 66
_NG = 6

_DEST = np.zeros((_NG, 16), dtype=np.int32)
_used = np.zeros(_IN_W, dtype=bool)
_used[_DIMS] = True
_off = 0
for _k in range(_NG):
    _u = 0
    for _l in range(16):
        if _used[_k * 16 + _l]:
            _DEST[_k, _l] = _off
            _off += 1
        else:
            _DEST[_k, _l] = _OUT_W + _u
            _u += 1
        assert _u <= 10
assert _off == _OUT_W

_ROWS_PER_CHUNK = 256
_NUM_WORKERS = 32


def _sc_kernel(n_rows: int):
    rows_per_worker = n_rows // _NUM_WORKERS
    n_chunks = rows_per_worker // _ROWS_PER_CHUNK
    assert rows_per_worker % _ROWS_PER_CHUNK == 0
    in_chunk = _ROWS_PER_CHUNK * _IN_W
    out_chunk = _ROWS_PER_CHUNK * _OUT_W
    mesh = plsc.VectorSubcoreMesh(core_axis_name="c", subcore_axis_name="s")

    @functools.partial(
        pl.kernel,
        out_type=jax.ShapeDtypeStruct((n_rows * _OUT_W,), jnp.float32),
        mesh=mesh,
        scratch_types=[
            pltpu.VMEM((in_chunk,), jnp.float32),
            pltpu.VMEM((out_chunk + 16,), jnp.float32),
            pltpu.VMEM((_IN_W,), jnp.float32),
            pltpu.VMEM((_IN_W,), jnp.float32),
            pltpu.VMEM((_NG * 16,), jnp.int32),
        ],
        compiler_params=pltpu.CompilerParams(needs_layout_passes=False),
    )
    def body(x_hbm, m_hbm, s_hbm, d_hbm, out_hbm, inbuf, outbuf, mbuf,
             sbuf, dbuf):
        wid = lax.axis_index("s") * mesh.num_cores + lax.axis_index("c")
        pltpu.sync_copy(m_hbm, mbuf)
        pltpu.sync_copy(s_hbm, sbuf)
        pltpu.sync_copy(d_hbm, dbuf)
        mv = [mbuf[pl.ds(16 * k, 16)] for k in range(_NG)]
        sv = [sbuf[pl.ds(16 * k, 16)] for k in range(_NG)]
        dv = tuple(dbuf[pl.ds(16 * k, 16)] for k in range(_NG))
        row0 = wid * rows_per_worker

        def chunk_body(c, _):
            base = row0 + c * _ROWS_PER_CHUNK
            pltpu.sync_copy(x_hbm.at[pl.ds(base * _IN_W, in_chunk)], inbuf)

            def row_body(r, idxs):
                rin = r * _IN_W
                nxt = []
                for k in range(_NG):
                    v = inbuf[pl.ds(rin + 16 * k, 16)]
                    v = (v - mv[k]) * sv[k]
                    plsc.store_scatter(outbuf, [idxs[k]], v)
                    nxt.append(idxs[k] + _OUT_W)
                return tuple(nxt)

            lax.fori_loop(0, _ROWS_PER_CHUNK, row_body, dv)
            pltpu.sync_copy(outbuf.at[pl.ds(0, out_chunk)],
                            out_hbm.at[pl.ds(base * _OUT_W, out_chunk)])
            return 0

        lax.fori_loop(0, n_chunks, chunk_body, 0)

    return body


@jax.jit
def kernel(observed_pose, mean, std):
    b, t, w = observed_pose.shape
    n_rows = b * t
    x = observed_pose.reshape(n_rows * w)
    dims = jnp.asarray(_DIMS, dtype=jnp.int32)
    m96 = jnp.zeros((_IN_W,), jnp.float32).at[dims].set(mean.reshape(-1))
    rs96 = jnp.zeros((_IN_W,), jnp.float32).at[dims].set(
        1.0 / std.reshape(-1))
    dtab = jnp.asarray(_DEST.reshape(-1))
    out_flat = _sc_kernel(n_rows)(x, m96, rs96, dtab)
    return out_flat.reshape(b, t, _OUT_W)


# speedup vs baseline: 2.4647x; 2.4647x over previous
import functools

import jax
import jax.numpy as jnp
import numpy as np
from jax.experimental import pallas as pl
from jax.experimental.pallas import tpu as pltpu

_DIMS = np.array(
    [6, 7, 8, 9, 10, 11, 12, 13, 14, 15, 16, 17, 21, 22, 23, 24, 25, 26,
     27, 28, 29, 30, 31, 32, 36, 37, 38, 39, 40, 41, 42, 43, 44, 45, 46,
     47, 51, 52, 53, 54, 55, 56, 57, 58, 59, 63, 64, 65, 66, 67, 68, 75,
     76, 77, 78, 79, 80, 81, 82, 83, 87, 88, 89, 90, 91, 92],
    dtype=np.int32,
)
_IN_W = 96
_OUT_W = 66
_NB = 128


def _tc_body(x_ref, s_ref, bi_ref, o_ref):
    nb, t, w = x_ref.shape
    x2 = x_ref[...].reshape(nb * t, w)
    g = jax.lax.dot_general(
        x2, s_ref[...], (((1,), (0,)), ((), ())),
        precision=jax.lax.Precision.DEFAULT,
        preferred_element_type=jnp.float32)
    o_ref[...] = (g + bi_ref[...]).reshape(nb, t, _OUT_W)


@jax.jit
def kernel(observed_pose, mean, std):
    b, t, w = observed_pose.shape
    scale = (1.0 / std).reshape(_OUT_W)
    bias = (-mean / std).reshape(1, _OUT_W)
    sel = jnp.zeros((_IN_W, _OUT_W), jnp.float32).at[
        jnp.asarray(_DIMS), jnp.arange(_OUT_W)].set(scale)
    grid = (b // _NB,)
    out = pl.pallas_call(
        _tc_body,
        grid=grid,
        in_specs=[
            pl.BlockSpec((_NB, t, w), lambda i: (i, 0, 0)),
            pl.BlockSpec((_IN_W, _OUT_W), lambda i: (0, 0)),
            pl.BlockSpec((1, _OUT_W), lambda i: (0, 0)),
        ],
        out_specs=pl.BlockSpec((_NB, t, _OUT_W), lambda i: (i, 0, 0)),
        out_shape=jax.ShapeDtypeStruct((b, t, _OUT_W), jnp.float32),
        compiler_params=pltpu.CompilerParams(
            dimension_semantics=("arbitrary",)),
    )(observed_pose, sel, bias)
    return out


# speedup vs baseline: 11.1069x; 4.5064x over previous
import jax
import jax.numpy as jnp
import numpy as np
from jax.experimental import pallas as pl
from jax.experimental.pallas import tpu as pltpu

_DIMS = np.array(
    [6, 7, 8, 9, 10, 11, 12, 13, 14, 15, 16, 17, 21, 22, 23, 24, 25, 26,
     27, 28, 29, 30, 31, 32, 36, 37, 38, 39, 40, 41, 42, 43, 44, 45, 46,
     47, 51, 52, 53, 54, 55, 56, 57, 58, 59, 63, 64, 65, 66, 67, 68, 75,
     76, 77, 78, 79, 80, 81, 82, 83, 87, 88, 89, 90, 91, 92],
    dtype=np.int32,
)
_IN_W = 96
_OUT_W = 66
_NBL = 4096


def _tc_body(x_ref, st_ref, bi_ref, o_ref):
    xs = x_ref[0]
    g = jax.lax.dot_general(
        st_ref[...], xs, (((1,), (0,)), ((), ())),
        precision=jax.lax.Precision.DEFAULT,
        preferred_element_type=jnp.float32)
    o_ref[...] = (g + bi_ref[...][:, 0:1])[None]


@jax.jit
def kernel(observed_pose, mean, std):
    b, t, w = observed_pose.shape
    xt = jnp.transpose(observed_pose, (1, 2, 0))
    scale = (1.0 / std).reshape(_OUT_W)
    bias = ((-mean / std).reshape(_OUT_W, 1) *
            jnp.ones((1, 128), jnp.float32))
    sel_t = jnp.zeros((_OUT_W, _IN_W), jnp.float32).at[
        jnp.arange(_OUT_W), jnp.asarray(_DIMS)].set(scale)
    grid = (t, b // _NBL)
    out_t = pl.pallas_call(
        _tc_body,
        grid=grid,
        in_specs=[
            pl.BlockSpec((1, w, _NBL), lambda ti, bi: (ti, 0, bi)),
            pl.BlockSpec((_OUT_W, _IN_W), lambda ti, bi: (0, 0)),
            pl.BlockSpec((_OUT_W, 128), lambda ti, bi: (0, 0)),
        ],
        out_specs=pl.BlockSpec((1, _OUT_W, _NBL), lambda ti, bi: (ti, 0, bi)),
        out_shape=jax.ShapeDtypeStruct((t, _OUT_W, b), jnp.float32),
        compiler_params=pltpu.CompilerParams(
            dimension_semantics=("arbitrary", "arbitrary")),
    )(xt, sel_t, bias)
    return jnp.transpose(out_t, (2, 0, 1))
